# Initial kernel scaffold; baseline (speedup 1.0000x reference)
import jax
import jax.numpy as jnp
from jax.experimental import pallas as pl


def kernel(population, target_prob):
    raise NotImplementedError("write your pallas kernel here")


# speedup vs baseline: 24.9611x; 24.9611x over previous
import jax
import jax.numpy as jnp
from jax import lax
from jax.experimental import pallas as pl
from jax.experimental.pallas import tpu as pltpu
from jax.experimental.pallas import tpu_sc as plsc

POP = 4096
PLEN = 2048
GRID = 1024
NCELL = GRID * GRID

NC = 2
NS = 16
L = 16

ROWS_PER_SC = POP // NC
CHUNK = PLEN // NS
NV = CHUNK // L
XBUF = CHUNK + 8
ZB = ROWS_PER_SC


def _body(xs_hbm, ys_hbm, tp_hbm, out_hbm,
          hist, oacc,
          xs_v, ys_v, ids_v, ones_v, cnts_v, probs_v, ridx_v, rval_v, zb_v,
          gsem):
    cid = lax.axis_index("c")
    sid = lax.axis_index("s")
    zeros16 = jnp.zeros((L,), jnp.float32)
    ones16 = jnp.ones((L,), jnp.float32)
    lane = lax.iota(jnp.int32, L)

    @pl.loop(0, ZB, step=L)
    def _(i):
        zb_v[pl.ds(i, L)] = zeros16

    for i in range(NV):
        ones_v[pl.ds(i * L, L)] = ones16

    hwords = NCELL // NS
    base = sid * hwords

    @pl.loop(0, hwords, step=ZB)
    def _(off):
        pltpu.sync_copy(zb_v, hist.at[pl.ds(base + off, ZB)])

    @pl.when(sid == 0)
    def _():
        pltpu.sync_copy(zb_v, oacc)

    plsc.subcore_barrier()

    @pl.loop(0, ROWS_PER_SC)
    def _(r):
        row = cid * ROWS_PER_SC + r
        start = row * PLEN + sid * CHUNK

        @pl.when(sid < NS - 1)
        def _():
            pltpu.sync_copy(xs_hbm.at[pl.ds(start, XBUF)],
                            xs_v.at[pl.ds(0, XBUF)])
            pltpu.sync_copy(ys_hbm.at[pl.ds(start, XBUF)],
                            ys_v.at[pl.ds(0, XBUF)])

        @pl.when(sid == NS - 1)
        def _():
            pltpu.sync_copy(xs_hbm.at[pl.ds(start, CHUNK)],
                            xs_v.at[pl.ds(0, CHUNK)])
            pltpu.sync_copy(ys_hbm.at[pl.ds(start, CHUNK)],
                            ys_v.at[pl.ds(0, CHUNK)])

        @pl.loop(0, CHUNK, step=L)
        def _(i):
            ids_v[pl.ds(i, L)] = xs_v[pl.ds(i, L)] * GRID + ys_v[pl.ds(i, L)]

        pgather = pltpu.async_copy(tp_hbm.at[ids_v], probs_v, gsem)

        pltpu.sync_copy(ones_v, hist.at[ids_v], add=True)
        plsc.subcore_barrier()

        pltpu.sync_copy(hist.at[ids_v], cnts_v)
        plsc.subcore_barrier()

        pltpu.sync_copy(zb_v.at[pl.ds(0, CHUNK)], hist.at[ids_v])

        pgather.wait()

        acc = jnp.zeros((L,), jnp.float32)
        for i in range(NV):
            o = i * L
            p = probs_v[pl.ds(o, L)]
            c = cnts_v[pl.ds(o, L)]
            rep = jnp.where(c >= 2.0, 1.0 / c, 0.0)
            xc = xs_v[pl.ds(o, L)]
            yc = ys_v[pl.ds(o, L)]
            xn = xs_v[pl.ds(o + 1, L)]
            yn = ys_v[pl.ds(o + 1, L)]
            d = jnp.abs(xn - xc) + jnp.abs(yn - yc)
            cont = jnp.where(d > 1, 0.5, 0.0)
            if i == NV - 1:
                tilefac = jnp.where(sid == NS - 1, 0.0, 1.0)
                cont = cont * jnp.where(lane == L - 1, tilefac, 1.0)
            acc = acc + p - 0.2 * rep - cont

        ridx_v[...] = jnp.full((L,), r, jnp.int32)
        rval_v[...] = acc
        pltpu.sync_copy(rval_v, oacc.at[ridx_v], add=True)
        plsc.subcore_barrier()

    @pl.when(sid == 0)
    def _():
        pltpu.sync_copy(oacc, out_hbm.at[pl.ds(cid * ROWS_PER_SC, ROWS_PER_SC)])


_sc_fitness = pl.kernel(
    _body,
    out_type=jax.ShapeDtypeStruct((POP,), jnp.float32),
    mesh=plsc.VectorSubcoreMesh(
        core_axis_name="c", subcore_axis_name="s",
        num_cores=NC, num_subcores=NS,
    ),
    scratch_types=[
        pltpu.VMEM_SHARED((NCELL,), jnp.float32),
        pltpu.VMEM_SHARED((ROWS_PER_SC,), jnp.float32),
        pltpu.VMEM((XBUF,), jnp.int32),
        pltpu.VMEM((XBUF,), jnp.int32),
        pltpu.VMEM((CHUNK,), jnp.int32),
        pltpu.VMEM((CHUNK,), jnp.float32),
        pltpu.VMEM((CHUNK,), jnp.float32),
        pltpu.VMEM((CHUNK,), jnp.float32),
        pltpu.VMEM((L,), jnp.int32),
        pltpu.VMEM((L,), jnp.float32),
        pltpu.VMEM((ZB,), jnp.float32),
        pltpu.SemaphoreType.DMA,
    ],
)


def kernel(population, target_prob):
    xs = population[:, :, 0].astype(jnp.int32).reshape(POP * PLEN)
    ys = population[:, :, 1].astype(jnp.int32).reshape(POP * PLEN)
    tp = target_prob.reshape(NCELL)
    return _sc_fitness(xs, ys, tp)
